# baseline (speedup 1.0000x reference)
import functools
import math

import jax
import jax.numpy as jnp
from jax import lax
from jax.experimental import pallas as pl
from jax.experimental.pallas import tpu as pltpu
from jax.experimental.pallas import tpu_sc as plsc

M_BLK = 512
K_BLK = 8192
K_SUB = 1024

_C = 64
_MU64 = math.sqrt(2.0) * math.exp(math.lgamma((_C + 1) / 2.0) - math.lgamma(_C / 2.0))

NUM_CORES = 2
NUM_SUBCORES = 16
NUM_WORKERS = NUM_CORES * NUM_SUBCORES
IDX_CHUNK = 128


def _match_body(z_ref, cbt_ref, idx_ref, mrg_ref):
    zb = z_ref[...]
    bv1 = bv2 = bargf = None
    for h in range(K_BLK // K_SUB):
        scores = jnp.dot(zb, cbt_ref[:, h * K_SUB:(h + 1) * K_SUB],
                         preferred_element_type=jnp.float32)
        v1 = jnp.max(scores, axis=1, keepdims=True)
        colf = (jax.lax.broadcasted_iota(jnp.int32, scores.shape, 1)
                + h * K_SUB).astype(jnp.float32)
        maskedf = jnp.where(scores == v1, colf, jnp.inf)
        argf = jnp.min(maskedf, axis=1, keepdims=True)
        v2 = jnp.max(jnp.where(colf == argf, -jnp.inf, scores), axis=1,
                     keepdims=True)
        if h == 0:
            bv1, bv2, bargf = v1, v2, argf
        else:
            upd = v1 > bv1
            bv2 = jnp.maximum(jnp.minimum(v1, bv1), jnp.maximum(v2, bv2))
            bv1 = jnp.where(upd, v1, bv1)
            bargf = jnp.where(upd, argf, bargf)
    idx_ref[...] = bargf.astype(jnp.int32)
    mrg_ref[...] = bv1 - bv2


def _match_argmax(z, cbt):
    m = z.shape[0]
    grid = (m // M_BLK,)
    return pl.pallas_call(
        _match_body,
        grid=grid,
        in_specs=[
            pl.BlockSpec((M_BLK, z.shape[1]), lambda i: (i, 0)),
            pl.BlockSpec((cbt.shape[0], K_BLK), lambda i: (0, 0)),
        ],
        out_specs=[
            pl.BlockSpec((M_BLK, 1), lambda i: (i, 0)),
            pl.BlockSpec((M_BLK, 1), lambda i: (i, 0)),
        ],
        out_shape=[
            jax.ShapeDtypeStruct((m, 1), jnp.int32),
            jax.ShapeDtypeStruct((m, 1), jnp.float32),
        ],
    )(z, cbt)


def _sc_gather(table, idx3d, n_rows, c):
    chunks = idx3d.shape[1]
    width = table.shape[1]
    mesh = plsc.VectorSubcoreMesh(core_axis_name="c", subcore_axis_name="s")

    @functools.partial(
        pl.kernel,
        out_type=jax.ShapeDtypeStruct((n_rows, width), jnp.float32),
        mesh=mesh,
        scratch_types=[
            pltpu.VMEM((chunks, IDX_CHUNK), jnp.int32),
            pltpu.VMEM((2, IDX_CHUNK, width), jnp.float32),
            pltpu.SemaphoreType.DMA,
        ],
    )
    def gather_kernel(table_hbm, idx_hbm, out_hbm, idx_v, rows_v, sem):
        wid = lax.axis_index("s") * NUM_CORES + lax.axis_index("c")
        base = wid * chunks * IDX_CHUNK
        pltpu.sync_copy(idx_hbm.at[wid], idx_v)
        copies = [None] * chunks
        for j in range(chunks):
            copies[j] = pltpu.async_copy(
                table_hbm.at[idx_v.at[j]], rows_v.at[j % 2], sem)
            if j > 0:
                copies[j - 1].wait()
                pltpu.sync_copy(
                    rows_v.at[(j - 1) % 2],
                    out_hbm.at[pl.ds(base + (j - 1) * IDX_CHUNK, IDX_CHUNK)])
        copies[chunks - 1].wait()
        pltpu.sync_copy(
            rows_v.at[(chunks - 1) % 2],
            out_hbm.at[pl.ds(base + (chunks - 1) * IDX_CHUNK, IDX_CHUNK)])

    return gather_kernel(table, idx3d)


TAU = 5e-4
FIX_ROWS = 256


def _fixup(z, codebook, idx, margin):
    rows = jnp.nonzero(margin[:, 0] < TAU, size=FIX_ROWS, fill_value=0)[0]
    z_sel = z[rows]
    z2 = jnp.sum(z_sel * z_sel, axis=-1, keepdims=True)
    e2 = jnp.sum(codebook * codebook, axis=-1)[None, :]
    d2 = jnp.maximum(z2 + e2 - 2.0 * (z_sel @ codebook.T), 0.0)
    simi = (-jnp.sqrt(d2) - _MU64) / math.sqrt(_C - _MU64 ** 2)
    zsoft = jax.nn.softmax(simi, axis=-1)
    fixed = jnp.argmax(zsoft, axis=-1).astype(idx.dtype)
    return idx.at[rows, 0].set(fixed)


def kernel(input, codebook):
    b, n, c = input.shape
    m = b * n
    z = input.reshape(m, c)
    idx2, margin = _match_argmax(z, codebook.T)
    idx2 = _fixup(z, codebook, idx2, margin)
    idx3d = idx2.reshape(NUM_WORKERS, m // (NUM_WORKERS * IDX_CHUNK), IDX_CHUNK)
    table = jnp.pad(codebook, ((0, 0), (0, 128 - c)))
    quant = _sc_gather(table, idx3d, m, c)[:, :c]
    return (idx2.reshape(b, n), quant.reshape(b, n, c))


# speedup vs baseline: 1.0301x; 1.0301x over previous
import functools
import math

import jax
import jax.numpy as jnp
from jax import lax
from jax.experimental import pallas as pl
from jax.experimental.pallas import tpu as pltpu
from jax.experimental.pallas import tpu_sc as plsc

M_BLK = 1024
K_BLK = 8192
K_SUB = 1024

_C = 64
_MU64 = math.sqrt(2.0) * math.exp(math.lgamma((_C + 1) / 2.0) - math.lgamma(_C / 2.0))

NUM_CORES = 2
NUM_SUBCORES = 16
NUM_WORKERS = NUM_CORES * NUM_SUBCORES
IDX_CHUNK = 128


def _match_body(z_ref, cbt_ref, idx_ref, mrg_ref):
    zb = z_ref[...]
    bv1 = bv2 = bargf = None
    for h in range(K_BLK // K_SUB):
        scores = jnp.dot(zb, cbt_ref[:, h * K_SUB:(h + 1) * K_SUB],
                         preferred_element_type=jnp.float32)
        v1 = jnp.max(scores, axis=1, keepdims=True)
        colf = (jax.lax.broadcasted_iota(jnp.int32, scores.shape, 1)
                + h * K_SUB).astype(jnp.float32)
        maskedf = jnp.where(scores == v1, colf, jnp.inf)
        argf = jnp.min(maskedf, axis=1, keepdims=True)
        v2 = jnp.max(jnp.where(colf == argf, -jnp.inf, scores), axis=1,
                     keepdims=True)
        if h == 0:
            bv1, bv2, bargf = v1, v2, argf
        else:
            upd = v1 > bv1
            bv2 = jnp.maximum(jnp.minimum(v1, bv1), jnp.maximum(v2, bv2))
            bv1 = jnp.where(upd, v1, bv1)
            bargf = jnp.where(upd, argf, bargf)
    idx_ref[...] = bargf.astype(jnp.int32)
    mrg_ref[...] = bv1 - bv2


def _match_argmax(z, cbt):
    m = z.shape[0]
    grid = (m // M_BLK,)
    return pl.pallas_call(
        _match_body,
        grid=grid,
        in_specs=[
            pl.BlockSpec((M_BLK, z.shape[1]), lambda i: (i, 0)),
            pl.BlockSpec((cbt.shape[0], K_BLK), lambda i: (0, 0)),
        ],
        out_specs=[
            pl.BlockSpec((M_BLK, 1), lambda i: (i, 0)),
            pl.BlockSpec((M_BLK, 1), lambda i: (i, 0)),
        ],
        out_shape=[
            jax.ShapeDtypeStruct((m, 1), jnp.int32),
            jax.ShapeDtypeStruct((m, 1), jnp.float32),
        ],
    )(z, cbt)


def _sc_gather(table, idx3d, n_rows, c):
    chunks = idx3d.shape[1]
    width = table.shape[1]
    mesh = plsc.VectorSubcoreMesh(core_axis_name="c", subcore_axis_name="s")

    @functools.partial(
        pl.kernel,
        out_type=jax.ShapeDtypeStruct((n_rows, width), jnp.float32),
        mesh=mesh,
        scratch_types=[
            pltpu.VMEM((chunks, IDX_CHUNK), jnp.int32),
            pltpu.VMEM((2, IDX_CHUNK, width), jnp.float32),
            pltpu.SemaphoreType.DMA,
        ],
    )
    def gather_kernel(table_hbm, idx_hbm, out_hbm, idx_v, rows_v, sem):
        wid = lax.axis_index("s") * NUM_CORES + lax.axis_index("c")
        base = wid * chunks * IDX_CHUNK
        pltpu.sync_copy(idx_hbm.at[wid], idx_v)
        copies = [None] * chunks
        for j in range(chunks):
            copies[j] = pltpu.async_copy(
                table_hbm.at[idx_v.at[j]], rows_v.at[j % 2], sem)
            if j > 0:
                copies[j - 1].wait()
                pltpu.sync_copy(
                    rows_v.at[(j - 1) % 2],
                    out_hbm.at[pl.ds(base + (j - 1) * IDX_CHUNK, IDX_CHUNK)])
        copies[chunks - 1].wait()
        pltpu.sync_copy(
            rows_v.at[(chunks - 1) % 2],
            out_hbm.at[pl.ds(base + (chunks - 1) * IDX_CHUNK, IDX_CHUNK)])

    return gather_kernel(table, idx3d)


TAU = 5e-4
FIX_ROWS = 256


def _fixup(z, codebook, idx, margin):
    rows = jnp.nonzero(margin[:, 0] < TAU, size=FIX_ROWS, fill_value=0)[0]
    z_sel = z[rows]
    z2 = jnp.sum(z_sel * z_sel, axis=-1, keepdims=True)
    e2 = jnp.sum(codebook * codebook, axis=-1)[None, :]
    d2 = jnp.maximum(z2 + e2 - 2.0 * (z_sel @ codebook.T), 0.0)
    simi = (-jnp.sqrt(d2) - _MU64) / math.sqrt(_C - _MU64 ** 2)
    zsoft = jax.nn.softmax(simi, axis=-1)
    fixed = jnp.argmax(zsoft, axis=-1).astype(idx.dtype)
    return idx.at[rows, 0].set(fixed)


def kernel(input, codebook):
    b, n, c = input.shape
    m = b * n
    z = input.reshape(m, c)
    idx2, margin = _match_argmax(z, codebook.T)
    idx2 = _fixup(z, codebook, idx2, margin)
    idx3d = idx2.reshape(NUM_WORKERS, m // (NUM_WORKERS * IDX_CHUNK), IDX_CHUNK)
    table = jnp.pad(codebook, ((0, 0), (0, 128 - c)))
    quant = _sc_gather(table, idx3d, m, c)[:, :c]
    return (idx2.reshape(b, n), quant.reshape(b, n, c))
